# Initial kernel scaffold; baseline (speedup 1.0000x reference)
import jax
import jax.numpy as jnp
from jax.experimental import pallas as pl


def kernel(features, bn1_gamma, bn1_beta, bn1_mean, bn1_var, bn2_gamma, bn2_beta, bn2_mean, bn2_var, gcn_weight, gcn_bias, aifa1, aifa2, aifa3):
    raise NotImplementedError("write your pallas kernel here")


# speedup vs baseline: 45.0644x; 45.0644x over previous
import jax
import jax.numpy as jnp
from jax.experimental import pallas as pl
from jax.experimental.pallas import tpu as pltpu

_N = 512
_K = 102
_EPS = 1e-5


def _make_A(x, a0, a1, a2, eye):
    n = x.shape[0]
    xt = jnp.transpose(x)
    sq_col = jnp.sum(x * x, axis=1, keepdims=True)
    sq_row = jnp.sum(xt * xt, axis=0, keepdims=True)
    G = jnp.dot(x, xt, preferred_element_type=jnp.float32)
    d2 = jnp.maximum(sq_col + sq_row - 2.0 * G, 0.0)
    E = jnp.exp(d2 * (-1.0 / 9.0))
    bits = jax.lax.bitcast_convert_type(E, jnp.int32)

    def body(i, t):
        trial = t | (jnp.int32(1) << (30 - i))
        cnt = jnp.sum((bits >= trial).astype(jnp.float32), axis=1, keepdims=True)
        return jnp.where(cnt >= float(_K), trial, t)

    kth = jax.lax.fori_loop(0, 31, body, jnp.zeros((n, 1), jnp.int32))

    mask = (bits >= kth).astype(jnp.float32) * (1.0 - eye)
    mask = mask * jnp.transpose(mask)
    adj = eye + mask * E
    deg_col = jnp.sum(adj, axis=1, keepdims=True) + 1.0
    deg_row = jnp.sum(adj, axis=0, keepdims=True) + 1.0
    An = adj * (1.0 / jnp.sqrt(deg_col)) * (1.0 / jnp.sqrt(deg_row))
    An2 = jnp.dot(An, An, preferred_element_type=jnp.float32)
    return a0 * eye + a1 * An + a2 * An2


def _fused(feat_ref, g1_ref, b1_ref, m1_ref, v1_ref,
           g2_ref, b2_ref, m2_ref, v2_ref,
           w_ref, bias_ref, aifa_ref, out_ref):
    n = _N
    ri = jax.lax.broadcasted_iota(jnp.int32, (n, n), 0)
    ci = jax.lax.broadcasted_iota(jnp.int32, (n, n), 1)
    eye = (ri == ci).astype(jnp.float32)
    a0 = aifa_ref[0]
    a1 = aifa_ref[1]
    a2 = aifa_ref[2]

    feat = feat_ref[...]
    A = _make_A(feat, a0, a1, a2, eye)
    h = jnp.dot(A, feat, preferred_element_type=jnp.float32)
    x = (h - m1_ref[...]) / jnp.sqrt(v1_ref[...] + _EPS) * g1_ref[...] + b1_ref[...]
    x = jnp.maximum(x, 0.0)

    A = _make_A(x, a0, a1, a2, eye)
    support = jnp.dot(x, w_ref[...], preferred_element_type=jnp.float32)
    out = jnp.dot(A, support, preferred_element_type=jnp.float32) + bias_ref[...]
    out = (out - m2_ref[...]) / jnp.sqrt(v2_ref[...] + _EPS) * g2_ref[...] + b2_ref[...]
    out_ref[...] = jnp.maximum(out, 0.0)


def kernel(features, bn1_gamma, bn1_beta, bn1_mean, bn1_var,
           bn2_gamma, bn2_beta, bn2_mean, bn2_var,
           gcn_weight, gcn_bias, aifa1, aifa2, aifa3):
    hid = gcn_weight.shape[1]
    aifa = jax.nn.softmax(jnp.concatenate([aifa1, aifa2, aifa3], axis=0))
    return pl.pallas_call(
        _fused,
        out_shape=jax.ShapeDtypeStruct((_N, hid), jnp.float32),
        in_specs=[pl.BlockSpec(memory_space=pltpu.VMEM)] * 11
        + [pl.BlockSpec(memory_space=pltpu.SMEM)],
        out_specs=pl.BlockSpec(memory_space=pltpu.VMEM),
    )(features, bn1_gamma, bn1_beta, bn1_mean, bn1_var,
      bn2_gamma, bn2_beta, bn2_mean, bn2_var,
      gcn_weight, gcn_bias, aifa)


# speedup vs baseline: 57.2612x; 1.2707x over previous
import jax
import jax.numpy as jnp
from jax.experimental import pallas as pl
from jax.experimental.pallas import tpu as pltpu

_N = 512
_K = 102
_EPS = 1e-5


def _make_A(x, a0, a1, a2, eye):
    n = x.shape[0]
    xt = jnp.transpose(x)
    G = jnp.dot(x, xt, preferred_element_type=jnp.float32)
    Gd = G * eye
    sq_col = jnp.sum(Gd, axis=1, keepdims=True)
    sq_row = jnp.sum(Gd, axis=0, keepdims=True)
    d2 = jnp.maximum(sq_col + sq_row - 2.0 * G, 0.0)
    E = jnp.exp(d2 * (-1.0 / 9.0))
    bits = jax.lax.bitcast_convert_type(E, jnp.int32)

    def body(i, t):
        trial = t | (jnp.int32(1) << (29 - i))
        cnt = jnp.sum((bits >= trial).astype(jnp.float32), axis=0, keepdims=True)
        return jnp.where(cnt >= float(_K), trial, t)

    kth = jax.lax.fori_loop(0, 30, body, jnp.zeros((1, n), jnp.int32))

    mask = (bits >= kth).astype(jnp.float32) * (1.0 - eye)
    mask = mask * jnp.transpose(mask)
    adj = eye + mask * E
    deg_col = jnp.sum(adj, axis=1, keepdims=True) + 1.0
    deg_row = jnp.sum(adj, axis=0, keepdims=True) + 1.0
    An = adj * (1.0 / jnp.sqrt(deg_col)) * (1.0 / jnp.sqrt(deg_row))
    An2 = jnp.dot(An, An, preferred_element_type=jnp.float32)
    return a0 * eye + a1 * An + a2 * An2


def _fused(feat_ref, g1_ref, b1_ref, m1_ref, v1_ref,
           g2_ref, b2_ref, m2_ref, v2_ref,
           w_ref, bias_ref, aifa_ref, out_ref):
    n = _N
    ri = jax.lax.broadcasted_iota(jnp.int32, (n, n), 0)
    ci = jax.lax.broadcasted_iota(jnp.int32, (n, n), 1)
    eye = (ri == ci).astype(jnp.float32)
    a0 = aifa_ref[0]
    a1 = aifa_ref[1]
    a2 = aifa_ref[2]

    feat = feat_ref[...]
    A = _make_A(feat, a0, a1, a2, eye)
    h = jnp.dot(A, feat, preferred_element_type=jnp.float32)
    x = (h - m1_ref[...]) / jnp.sqrt(v1_ref[...] + _EPS) * g1_ref[...] + b1_ref[...]
    x = jnp.maximum(x, 0.0)

    A = _make_A(x, a0, a1, a2, eye)
    support = jnp.dot(x, w_ref[...], preferred_element_type=jnp.float32)
    out = jnp.dot(A, support, preferred_element_type=jnp.float32) + bias_ref[...]
    out = (out - m2_ref[...]) / jnp.sqrt(v2_ref[...] + _EPS) * g2_ref[...] + b2_ref[...]
    out_ref[...] = jnp.maximum(out, 0.0)


def kernel(features, bn1_gamma, bn1_beta, bn1_mean, bn1_var,
           bn2_gamma, bn2_beta, bn2_mean, bn2_var,
           gcn_weight, gcn_bias, aifa1, aifa2, aifa3):
    hid = gcn_weight.shape[1]
    aifa = jax.nn.softmax(jnp.concatenate([aifa1, aifa2, aifa3], axis=0))
    return pl.pallas_call(
        _fused,
        out_shape=jax.ShapeDtypeStruct((_N, hid), jnp.float32),
        in_specs=[pl.BlockSpec(memory_space=pltpu.VMEM)] * 11
        + [pl.BlockSpec(memory_space=pltpu.SMEM)],
        out_specs=pl.BlockSpec(memory_space=pltpu.VMEM),
    )(features, bn1_gamma, bn1_beta, bn1_mean, bn1_var,
      bn2_gamma, bn2_beta, bn2_mean, bn2_var,
      gcn_weight, gcn_bias, aifa)
